# baseline (speedup 1.0000x reference)
import functools

import jax
import jax.numpy as jnp
from jax import lax
from jax.experimental import pallas as pl
from jax.experimental.pallas import tpu as pltpu
from jax.experimental.pallas import tpu_sc as plsc

NBUF = 2


@functools.lru_cache(maxsize=None)
def _make_gather_kernel(B, S, D, V):
    info = plsc.get_sparse_core_info()
    NC, NS = info.num_cores, info.num_subcores
    NW = NC * NS
    BB = B // 128
    nq = S * BB
    assert B % 128 == 0 and nq % (NW * NBUF) == 0 and V % 2 == 0
    niter = nq // (NW * NBUF)
    W = 2 * D
    mesh = plsc.VectorSubcoreMesh(core_axis_name="c", subcore_axis_name="s")

    @functools.partial(
        pl.kernel,
        mesh=mesh,
        out_type=jax.ShapeDtypeStruct((B * S, W), jnp.float32),
        scratch_types=[
            [pltpu.VMEM((128,), jnp.int32)] * NBUF,
            [pltpu.VMEM((128, W), jnp.float32)] * NBUF,
            [pltpu.SemaphoreType.DMA] * NBUF,
            [pltpu.SemaphoreType.DMA] * NBUF,
            [pltpu.SemaphoreType.DMA] * NBUF,
        ],
    )
    def gather_kernel(idxh_hbm, wtep_hbm, mid_hbm,
                      idx_v, rows_v, i_sems, g_sems, o_sems):
        wid = lax.axis_index("s") * NC + lax.axis_index("c")

        def start_idx(q, b):
            return pltpu.async_copy(
                idxh_hbm.at[pl.ds(q * 128, 128)], idx_v[b], i_sems[b])

        def wait_idx(b):
            pltpu.make_async_copy(
                idxh_hbm.at[pl.ds(0, 128)], idx_v[b], i_sems[b]).wait()

        def start_gather(b):
            return pltpu.async_copy(
                wtep_hbm.at[idx_v[b]], rows_v[b], g_sems[b])

        def wait_gather(b):
            pltpu.make_async_copy(
                wtep_hbm.at[idx_v[b]], rows_v[b], g_sems[b]).wait()

        def wait_out(b):
            pltpu.make_async_copy(
                rows_v[b], mid_hbm.at[pl.ds(0, 128)], o_sems[b]).wait()

        for b in range(NBUF):
            start_idx(wid * NBUF + b, b)
        for b in range(NBUF):
            wait_idx(b)
            start_gather(b)

        def iter_body(i, _):
            q0 = i * NW * NBUF + wid * NBUF
            for b in range(NBUF):
                q = q0 + b
                wait_gather(b)

                @pl.when(i + 1 < niter)
                def _():
                    start_idx(q + NW * NBUF, b)

                @pl.when(i > 0)
                def _():
                    wait_out(b)

                pltpu.async_copy(
                    rows_v[b], mid_hbm.at[pl.ds(q * 128, 128)], o_sems[b])

                @pl.when(i + 1 < niter)
                def _():
                    wait_idx(b)
                    start_gather(b)

            return 0

        lax.fori_loop(0, niter, iter_body, 0)
        for b in range(NBUF):
            wait_out(b)

    return gather_kernel


@functools.lru_cache(maxsize=None)
def _make_finish_kernel(B, S, D):
    BB = B // 128
    DG = D // 8
    W = 2 * D

    def body(mid_ref, par_ref, wpe_ref, out_ref):
        x = mid_ref[...]
        p = par_ref[...].reshape(128, 1)
        y = jnp.where(p == 1, x[:, D:], x[:, :D])
        y = y + wpe_ref[0]
        out_ref[0, :, 0, :, :] = jnp.swapaxes(y, 0, 1).reshape(DG, 8, 128)

    return pl.pallas_call(
        body,
        grid=(S, BB),
        in_specs=[
            pl.BlockSpec((128, W), lambda s, bg: (s * BB + bg, 0)),
            pl.BlockSpec((1, 1, 128), lambda s, bg: (s * BB + bg, 0, 0)),
            pl.BlockSpec((1, 1, D), lambda s, bg: (s, 0, 0)),
        ],
        out_specs=pl.BlockSpec(
            (1, DG, 1, 8, 128), lambda s, bg: (s, 0, bg, 0, 0)),
        out_shape=jax.ShapeDtypeStruct((S, DG, BB, 8, 128), jnp.float32),
    )


def kernel(idx, wte, wpe):
    B, S = idx.shape
    V, D = wte.shape
    idxf = idx.T.reshape(-1)
    mid = _make_gather_kernel(B, S, D, V)(
        idxf >> 1, wte.reshape(V // 2, 2 * D))
    out4 = _make_finish_kernel(B, S, D)(
        mid, (idxf & 1).reshape(B * S // 128, 1, 128), wpe.reshape(S, 1, D))
    return out4.transpose(2, 4, 0, 1, 3).reshape(B, S, D)


# speedup vs baseline: 1.9539x; 1.9539x over previous
import functools

import jax
import jax.numpy as jnp
from jax import lax
from jax.experimental import pallas as pl
from jax.experimental.pallas import tpu as pltpu
from jax.experimental.pallas import tpu_sc as plsc

NBUF = 2


@functools.lru_cache(maxsize=None)
def _make_repack_kernel(V, D):
    CH = 8192
    H = CH // 2
    NCH = (V + CH - 1) // CH

    def body(wt_ref, out_ref):
        y = jnp.swapaxes(wt_ref[...], 0, 1)
        out_ref[:, :D] = y[:H]
        out_ref[:, D:] = y[H:]

    return pl.pallas_call(
        body,
        grid=(NCH,),
        in_specs=[pl.BlockSpec((D, CH), lambda c: (0, c))],
        out_specs=pl.BlockSpec((H, 2 * D), lambda c: (c, 0)),
        out_shape=jax.ShapeDtypeStruct((NCH * H, 2 * D), jnp.float32),
    )


@functools.lru_cache(maxsize=None)
def _make_gather_kernel(B, S, D, V):
    info = plsc.get_sparse_core_info()
    NC, NS = info.num_cores, info.num_subcores
    NW = NC * NS
    BB = B // 128
    nq = S * BB
    assert B % 128 == 0 and nq % (NW * NBUF) == 0 and V % 2 == 0
    niter = nq // (NW * NBUF)
    W = 2 * D
    mesh = plsc.VectorSubcoreMesh(core_axis_name="c", subcore_axis_name="s")

    @functools.partial(
        pl.kernel,
        mesh=mesh,
        out_type=jax.ShapeDtypeStruct((B * S, W), jnp.float32),
        scratch_types=[
            [pltpu.VMEM((128,), jnp.int32)] * NBUF,
            [pltpu.VMEM((128, W), jnp.float32)] * NBUF,
            [pltpu.SemaphoreType.DMA] * NBUF,
            [pltpu.SemaphoreType.DMA] * NBUF,
            [pltpu.SemaphoreType.DMA] * NBUF,
        ],
    )
    def gather_kernel(idxh_hbm, wtep_hbm, mid_hbm,
                      idx_v, rows_v, i_sems, g_sems, o_sems):
        wid = lax.axis_index("s") * NC + lax.axis_index("c")

        def start_idx(q, b):
            return pltpu.async_copy(
                idxh_hbm.at[pl.ds(q * 128, 128)], idx_v[b], i_sems[b])

        def wait_idx(b):
            pltpu.make_async_copy(
                idxh_hbm.at[pl.ds(0, 128)], idx_v[b], i_sems[b]).wait()

        def start_gather(b):
            return pltpu.async_copy(
                wtep_hbm.at[idx_v[b]], rows_v[b], g_sems[b])

        def wait_gather(b):
            pltpu.make_async_copy(
                wtep_hbm.at[idx_v[b]], rows_v[b], g_sems[b]).wait()

        def wait_out(b):
            pltpu.make_async_copy(
                rows_v[b], mid_hbm.at[pl.ds(0, 128)], o_sems[b]).wait()

        for b in range(NBUF):
            start_idx(wid * NBUF + b, b)
        for b in range(NBUF):
            wait_idx(b)
            start_gather(b)

        def iter_body(i, _):
            q0 = i * NW * NBUF + wid * NBUF
            for b in range(NBUF):
                q = q0 + b
                wait_gather(b)

                @pl.when(i + 1 < niter)
                def _():
                    start_idx(q + NW * NBUF, b)

                @pl.when(i > 0)
                def _():
                    wait_out(b)

                pltpu.async_copy(
                    rows_v[b], mid_hbm.at[pl.ds(q * 128, 128)], o_sems[b])

                @pl.when(i + 1 < niter)
                def _():
                    wait_idx(b)
                    start_gather(b)

            return 0

        lax.fori_loop(0, niter, iter_body, 0)
        for b in range(NBUF):
            wait_out(b)

    return gather_kernel


@functools.lru_cache(maxsize=None)
def _make_finish_kernel(B, S, D):
    BB = B // 128
    W = 2 * D
    SB = 8

    def body(mid_ref, par_ref, wpe_ref, out_ref):
        for sp in range(SB):
            x = mid_ref[sp, 0]
            p = par_ref[sp, 0, 0].reshape(128, 1)
            y = jnp.where(p == 1, x[:, D:], x[:, :D])
            out_ref[:, sp, :] = y + wpe_ref[sp]

    return pl.pallas_call(
        body,
        grid=(S // SB, BB),
        in_specs=[
            pl.BlockSpec((SB, 1, 128, W), lambda so, bg: (so, bg, 0, 0)),
            pl.BlockSpec((SB, 1, 1, 128), lambda so, bg: (so, bg, 0, 0)),
            pl.BlockSpec((SB, D), lambda so, bg: (so, 0)),
        ],
        out_specs=pl.BlockSpec((128, SB, D), lambda so, bg: (bg, so, 0)),
        out_shape=jax.ShapeDtypeStruct((B, S, D), jnp.float32),
    )


def kernel(idx, wte, wpe):
    B, S = idx.shape
    V, D = wte.shape
    BB = B // 128
    wtep = _make_repack_kernel(V, D)(wte.T)
    VP = wtep.shape[0]
    idxf = idx.T.reshape(-1)
    idxp = ((idxf >> 13) << 12) | (idxf & 4095)
    half = (idxf >> 12) & 1
    mid = _make_gather_kernel(B, S, D, VP * 2)(idxp, wtep)
    return _make_finish_kernel(B, S, D)(
        mid.reshape(S, BB, 128, 2 * D),
        half.reshape(S, BB, 1, 128),
        wpe,
    )


# speedup vs baseline: 2.0614x; 1.0550x over previous
import functools

import jax
import jax.numpy as jnp
from jax import lax
from jax.experimental import pallas as pl
from jax.experimental.pallas import tpu as pltpu
from jax.experimental.pallas import tpu_sc as plsc

LANES = 16
NBUF = 4
AHEAD = 2


@functools.lru_cache(maxsize=None)
def _make_emb_kernel(B, S, D, V):
    N = B * S
    info = plsc.get_sparse_core_info()
    NC, NS = info.num_cores, info.num_subcores
    NW = NC * NS
    assert B % NW == 0, (B, NW)
    nchunks = B // NW
    mesh = plsc.VectorSubcoreMesh(core_axis_name="c", subcore_axis_name="s")

    @functools.partial(
        pl.kernel,
        mesh=mesh,
        compiler_params=pltpu.CompilerParams(use_tc_tiling_on_sc=False),
        out_type=jax.ShapeDtypeStruct((N, D), jnp.float32),
        scratch_types=[
            pltpu.VMEM((nchunks, S), jnp.int32),
            pltpu.VMEM((S, D), jnp.float32),
            [pltpu.VMEM((S, D), jnp.float32)] * NBUF,
            [pltpu.SemaphoreType.DMA] * NBUF,
            [pltpu.SemaphoreType.DMA] * NBUF,
        ],
    )
    def emb_kernel(idx_hbm, wte_hbm, wpe_hbm, out_hbm,
                   idx_v, wpe_v, rows_v, g_sems, o_sems):
        wid = lax.axis_index("s") * NC + lax.axis_index("c")
        row0 = wid * nchunks
        pltpu.sync_copy(wpe_hbm, wpe_v)
        pltpu.sync_copy(idx_hbm.at[pl.ds(row0, nchunks)], idx_v)

        def start_gather(c):
            b = c % NBUF
            return pltpu.async_copy(
                wte_hbm.at[idx_v.at[c]], rows_v[b], g_sems[b])

        g_handles = {}
        o_handles = {}
        for c in range(min(AHEAD, nchunks)):
            g_handles[c] = start_gather(c)

        for c in range(nchunks):
            b = c % NBUF
            ca = c + AHEAD
            if ca < nchunks:
                if ca >= NBUF:
                    o_handles[ca - NBUF].wait()
                g_handles[ca] = start_gather(ca)
            g_handles[c].wait()

            @plsc.parallel_loop(0, S, 2, unroll=4)
            def _(r):
                for k in range(2):
                    for j in range(D // LANES):
                        plsc.addupdate(
                            rows_v[b].at[r + k, pl.ds(j * LANES, LANES)],
                            wpe_v[r + k, pl.ds(j * LANES, LANES)],
                        )

            o_handles[c] = pltpu.async_copy(
                rows_v[b], out_hbm.at[pl.ds((row0 + c) * S, S)], o_sems[b])

        for c in range(max(0, nchunks - NBUF), nchunks):
            o_handles[c].wait()

    return emb_kernel


def kernel(idx, wte, wpe):
    B, S = idx.shape
    V, D = wte.shape
    emb = _make_emb_kernel(B, S, D, V)(idx, wte, wpe)
    return emb.reshape(B, S, D)
